# baseline (speedup 1.0000x reference)
import functools
import jax
import jax.numpy as jnp
from jax import lax
from jax.experimental import pallas as pl
from jax.experimental.pallas import tpu as pltpu
from jax.experimental.pallas import tpu_sc as plsc

K = 7
EPS = 1e-8
L = 16
H = W = 128
CE = 8
C = 3
NEG_BIG = -1.0e30
PADW = 176


def _sc_body(xeh, yeh, xph, lth, zh,
             xel, yel, xl, ltl,
             nx, bxv, bxb, nyb, byv, byb, ltv, itb,
             cc, vsb, ub, mxb, sb, outl):
    f32 = jnp.float32
    wid = lax.axis_index("s") * 2 + lax.axis_index("c")

    def chunk_body(t, _c):
        h0 = 2 * wid + 64 * t

        pltpu.sync_copy(xeh.at[pl.ds(h0, 18)], xel)
        pltpu.sync_copy(yeh.at[pl.ds(h0, 4)], yel)
        pltpu.sync_copy(xph.at[pl.ds(h0, 16)], xl)
        pltpu.sync_copy(lth.at[pl.ds(h0, 4)], ltl)

        def nx_row(row, _):
            def nx_col(kk, _2):
                b = kk * L
                a = xel[row, 0, pl.ds(b, L)]
                acc = a * a
                for c in range(1, CE):
                    v = xel[row, c, pl.ds(b, L)]
                    acc = acc + v * v
                nx[row, pl.ds(b, L)] = acc
                return 0
            lax.fori_loop(0, 11, nx_col, 0)
            return 0
        lax.fori_loop(0, 18, nx_row, 0)

        def bxv_row(row, _):
            def col(kk, _2):
                b = kk * L
                bxv[row, pl.ds(b, L)] = (nx[row, pl.ds(b, L)]
                                         + nx[row + 1, pl.ds(b, L)]
                                         + nx[row + 2, pl.ds(b, L)])
                return 0
            lax.fori_loop(0, 11, col, 0)
            return 0
        lax.fori_loop(0, 16, bxv_row, 0)

        def bx_row(row, _):
            def col(kk, _2):
                b = jnp.where(kk == 0, 8, kk * L)
                bxb[row, pl.ds(b, L)] = (bxv[row, pl.ds(b - 1, L)]
                                         + bxv[row, pl.ds(b, L)]
                                         + bxv[row, pl.ds(b + 1, L)])
                return 0
            lax.fori_loop(0, 10, col, 0)
            return 0
        lax.fori_loop(0, 16, bx_row, 0)

        def ny_row(row, _):
            def ny_col(kk, _2):
                b = kk * L
                a = yel[row, 0, pl.ds(b, L)]
                acc = a * a
                for c in range(1, CE):
                    v = yel[row, c, pl.ds(b, L)]
                    acc = acc + v * v
                nyb[row, pl.ds(b, L)] = acc
                return 0
            lax.fori_loop(0, 11, ny_col, 0)
            return 0
        lax.fori_loop(0, 4, ny_row, 0)

        def byv_col(kk, _):
            b = kk * L
            for rp in range(2):
                byv[rp, pl.ds(b, L)] = (nyb[rp, pl.ds(b, L)]
                                        + nyb[rp + 1, pl.ds(b, L)]
                                        + nyb[rp + 2, pl.ds(b, L)])
            return 0
        lax.fori_loop(0, 11, byv_col, 0)

        def ltv_col(kk, _):
            b = kk * L
            for rp in range(2):
                ltv[rp, pl.ds(b, L)] = (ltl[rp, pl.ds(b, L)]
                                        + ltl[rp + 1, pl.ds(b, L)]
                                        + ltl[rp + 2, pl.ds(b, L)])
            return 0
        lax.fori_loop(0, 11, ltv_col, 0)

        neg_init = jnp.full((L,), -3.0e38, f32)
        def grid_pre(vv, _):
            j0 = (vv + 1) * L
            b = vv * L
            for rp in range(2):
                byb[rp, pl.ds(j0, L)] = (byv[rp, pl.ds(j0 - 1, L)]
                                         + byv[rp, pl.ds(j0, L)]
                                         + byv[rp, pl.ds(j0 + 1, L)])
                blt = (ltv[rp, pl.ds(j0 - 1, L)]
                       + ltv[rp, pl.ds(j0, L)]
                       + ltv[rp, pl.ds(j0 + 1, L)])
                tv = jnp.exp(blt * (1.0 / 9.0))
                itb[rp, pl.ds(j0, L)] = -1.0 / (tv + EPS)
                mxb[rp, pl.ds(b, L)] = neg_init
            return 0
        lax.fori_loop(0, 8, grid_pre, 0)

        def dy_body(i, _):
            def cc_rr(rr, _2):
                def cc_kk(kk, _3):
                    b = 15 + kk * L
                    yv = [yel[rr, c, pl.ds(b, L)] for c in range(CE)]
                    def cc_dx(dxk, _4):
                        xo = b + dxk - 7
                        acc = yv[0] * xel[rr + i, 0, pl.ds(xo, L)]
                        for c in range(1, CE):
                            acc = acc + yv[c] * xel[rr + i, c, pl.ds(xo, L)]
                        cc[dxk, rr, pl.ds(b, L)] = acc
                        return 0
                    lax.fori_loop(0, 15, cc_dx, 0)
                    return 0
                lax.fori_loop(0, 9, cc_kk, 0)
                return 0
            lax.fori_loop(0, 4, cc_rr, 0)

            def b_dx(dxk, _2):
                def vs_kk(kk, _3):
                    b = 15 + kk * L
                    for rp in range(2):
                        vsb[rp, pl.ds(b, L)] = (cc[dxk, rp, pl.ds(b, L)]
                                                + cc[dxk, rp + 1, pl.ds(b, L)]
                                                + cc[dxk, rp + 2, pl.ds(b, L)])
                    return 0
                lax.fori_loop(0, 9, vs_kk, 0)

                o = i * 15 + dxk
                pen = jnp.where(o == 112, NEG_BIG, 0.0).astype(f32)
                def u_vv(vv, _3):
                    j0 = (vv + 1) * L
                    b = vv * L
                    for rp in range(2):
                        bc = (vsb[rp, pl.ds(j0 - 1, L)]
                              + vsb[rp, pl.ds(j0, L)]
                              + vsb[rp, pl.ds(j0 + 1, L)])
                        bxs = bxb[rp + i, pl.ds(j0 + dxk - 7, L)]
                        d = byb[rp, pl.ds(j0, L)] + bxs - 2.0 * bc
                        lg = d * itb[rp, pl.ds(j0, L)] + pen
                        ub[o, rp, pl.ds(b, L)] = lg
                        mxb[rp, pl.ds(b, L)] = jnp.maximum(
                            mxb[rp, pl.ds(b, L)], lg)
                    return 0
                lax.fori_loop(0, 8, u_vv, 0)
                return 0
            lax.fori_loop(0, 15, b_dx, 0)
            return 0
        lax.fori_loop(0, 15, dy_body, 0)

        def exp_o(o, _):
            def exp_vv(vv, _2):
                b = vv * L
                for rp in range(2):
                    ub[o, rp, pl.ds(b, L)] = jnp.exp(
                        ub[o, rp, pl.ds(b, L)] - mxb[rp, pl.ds(b, L)])
                return 0
            lax.fori_loop(0, 8, exp_vv, 0)
            return 0
        lax.fori_loop(0, 225, exp_o, 0)

        for jj in range(K):
            def s_vv(vv, _):
                b = vv * L
                for rp in range(2):
                    def s_oo(oo, s):
                        for ii in range(15):
                            s = s + ub[oo * 15 + ii, rp, pl.ds(b, L)]
                        return s
                    s = lax.fori_loop(0, 15, s_oo, jnp.zeros((L,), f32))
                    s = jnp.maximum(s, 1.0e-35)
                    sb[rp, pl.ds(b, L)] = 1.0 / s
                return 0
            lax.fori_loop(0, 8, s_vv, 0)

            def agg_vv(vv, _):
                b = vv * L
                j0 = (vv + 1) * L
                for rp in range(2):
                    invs = sb[rp, pl.ds(b, L)]
                    def agg_ii(ii, zz):
                        z0, z1, z2 = zz
                        xrow = rp + ii
                        for dxk in range(15):
                            o = ii * 15 + dxk
                            u = ub[o, rp, pl.ds(b, L)]
                            w = u * invs
                            co = j0 + dxk - 7
                            z0 = z0 + w * xl[xrow, 0, pl.ds(co, L)]
                            z1 = z1 + w * xl[xrow, 1, pl.ds(co, L)]
                            z2 = z2 + w * xl[xrow, 2, pl.ds(co, L)]
                            ub[o, rp, pl.ds(b, L)] = u * (1.0 + (EPS - w))
                        return (z0, z1, z2)
                    zz = lax.fori_loop(
                        0, 15, agg_ii,
                        (jnp.zeros((L,), f32), jnp.zeros((L,), f32),
                         jnp.zeros((L,), f32)))
                    for c in range(C):
                        outl[rp, jj * C + c, pl.ds(b, L)] = (
                            zz[c] - xl[rp + 7, c, pl.ds(j0, L)])
                return 0
            lax.fori_loop(0, 8, agg_vv, 0)

        pltpu.sync_copy(outl, zh.at[pl.ds(h0, 2)])
        return 0

    lax.fori_loop(0, 2, chunk_body, 0)


@jax.jit
def _run(x, xe, ye, log_temp):
    x0 = x[0]
    xe0 = xe[0]
    ye0 = ye[0]
    lt0 = log_temp[0, 0]

    xep = jnp.pad(xe0, ((0, 0), (8, 8), (16, 32)), mode="wrap")
    xep = xep.transpose(1, 0, 2)
    yep = jnp.pad(ye0, ((0, 0), (1, 1), (16, 32)), mode="wrap")
    yep = yep.transpose(1, 0, 2)
    xpp = jnp.pad(x0, ((0, 0), (7, 7), (16, 32)), mode="wrap")
    xpp = xpp.transpose(1, 0, 2)
    ltp = jnp.pad(lt0, ((1, 1), (16, 32)), mode="wrap")

    mesh = plsc.VectorSubcoreMesh(core_axis_name="c", subcore_axis_name="s")
    f32 = jnp.float32
    zk = pl.kernel(
        _sc_body,
        mesh=mesh,
        out_type=jax.ShapeDtypeStruct((H, K * C, W), f32),
        compiler_params=pltpu.CompilerParams(use_tc_tiling_on_sc=False),
        scratch_types=[
            pltpu.VMEM((18, CE, PADW), f32),
            pltpu.VMEM((4, CE, PADW), f32),
            pltpu.VMEM((16, C, PADW), f32),
            pltpu.VMEM((4, PADW), f32),
            pltpu.VMEM((18, PADW), f32),
            pltpu.VMEM((16, PADW), f32),
            pltpu.VMEM((16, PADW), f32),
            pltpu.VMEM((4, PADW), f32),
            pltpu.VMEM((2, PADW), f32),
            pltpu.VMEM((2, PADW), f32),
            pltpu.VMEM((2, PADW), f32),
            pltpu.VMEM((2, PADW), f32),
            pltpu.VMEM((15, 4, PADW), f32),
            pltpu.VMEM((2, PADW), f32),
            pltpu.VMEM((225, 2, W), f32),
            pltpu.VMEM((2, W), f32),
            pltpu.VMEM((2, W), f32),
            pltpu.VMEM((2, K * C, W), f32),
        ],
    )(xep, yep, xpp, ltp)

    z = zk.transpose(1, 0, 2)
    out = jnp.concatenate([x0, z], axis=0)[None]
    return out


def kernel(x, xe, ye, log_temp):
    return _run(x, xe, ye, log_temp)


# speedup vs baseline: 3.9152x; 3.9152x over previous
import functools
import jax
import jax.numpy as jnp
from jax import lax
from jax.experimental import pallas as pl
from jax.experimental.pallas import tpu as pltpu

K = 7
PS = 3
WS = 15
EPS = 1e-8
R = WS // 2
O = WS * WS
H = W = 128
CE = 8
C = 3
NEG_BIG = -1.0e30


def _rollw(a, s):
    if s % W == 0:
        return a
    return jnp.roll(a, s, axis=-1)


def _boxw(a):
    return a + _rollw(a, 1) + _rollw(a, -1)


def _nnn_kernel(xe2_ref, ye2_ref, x2_ref, lt_ref, out_ref, u_ref, bx_ref):
    f32 = jnp.float32

    lt = lt_ref[0]
    blt = _boxw(lt[0:H] + lt[1:H + 1] + lt[2:H + 2])
    invt = -1.0 / (jnp.exp(blt * (1.0 / (PS * PS))) + EPS)

    ny = jnp.zeros((H + 2, W), f32)
    for c in range(CE):
        yc = ye2_ref[c]
        ny = ny + yc * yc
    by = _boxw(ny[0:H] + ny[1:H + 1] + ny[2:H + 2])

    nx = jnp.zeros((H + 16, W), f32)
    for c in range(CE):
        xc = xe2_ref[c]
        nx = nx + xc * xc
    bx_ref[...] = _boxw(nx[0:H + 14] + nx[1:H + 15] + nx[2:H + 16])

    m = jnp.full((H, W), -3.0e38, f32)
    for dxk in range(WS):
        dx = dxk - R
        yv = [_rollw(ye2_ref[c], dx) for c in range(CE)]
        by_s = _rollw(by, dx)
        it_s = _rollw(invt, dx)

        def dy_body(i, mm, yv=yv, by_s=by_s, it_s=it_s, dxk=dxk, dx=dx):
            cc = jnp.zeros((H + 2, W), f32)
            for c in range(CE):
                cc = cc + yv[c] * xe2_ref[c, pl.ds(i, H + 2), :]
            bc = _boxw(cc[0:H] + cc[1:H + 1] + cc[2:H + 2])
            d = by_s + bx_ref[pl.ds(i, H), :] - 2.0 * bc
            lg = _rollw(d * it_s, -dx)
            if dxk == R:
                lg = jnp.where(i == R, NEG_BIG, lg)
            u_ref[pl.ds(i * WS + dxk, 1)] = lg[None]
            return jnp.maximum(mm, lg)

        m = lax.fori_loop(0, WS, dy_body, m)

    def exp_body(o, _):
        u_ref[pl.ds(o, 1)] = jnp.exp(u_ref[pl.ds(o, 1)] - m[None])
        return 0

    lax.fori_loop(0, O, exp_body, 0)

    def sum_body(o, s):
        return s + u_ref[pl.ds(o, 1)][0]

    for j in range(K):
        s = lax.fori_loop(0, O, sum_body, jnp.zeros((H, W), f32))

        z0 = jnp.zeros((H, W), f32)
        zacc = [z0, z0, z0]
        for dxk in range(WS):
            dx = dxk - R

            def agg_body(i, accs, dxk=dxk, dx=dx):
                o = i * WS + dxk
                u = u_ref[pl.ds(o, 1)][0]
                w = u / s
                u_ref[pl.ds(o, 1)] = (u * (1.0 + (EPS - w)))[None]
                ws = _rollw(w, dx)
                a0, a1, a2 = accs
                a0 = a0 + ws * x2_ref[0, pl.ds(i, H), :]
                a1 = a1 + ws * x2_ref[1, pl.ds(i, H), :]
                a2 = a2 + ws * x2_ref[2, pl.ds(i, H), :]
                return (a0, a1, a2)

            acc = lax.fori_loop(0, WS, agg_body, (z0, z0, z0))
            for c in range(C):
                zacc[c] = zacc[c] + _rollw(acc[c], -dx)
        for c in range(C):
            out_ref[j * C + c] = zacc[c] - x2_ref[c, pl.ds(R, H), :]


@jax.jit
def _run(x, xe, ye, log_temp):
    x0 = x[0]
    xe0 = xe[0]
    ye0 = ye[0]
    lt0 = log_temp[0]

    xe2 = jnp.concatenate([xe0[:, -8:, :], xe0, xe0[:, :8, :]], axis=1)
    ye2 = jnp.concatenate([ye0[:, -1:, :], ye0, ye0[:, :1, :]], axis=1)
    x2 = jnp.concatenate([x0[:, -R:, :], x0, x0[:, :R, :]], axis=1)
    lt2 = jnp.concatenate([lt0[:, -1:, :], lt0, lt0[:, :1, :]], axis=1)

    z = pl.pallas_call(
        _nnn_kernel,
        out_shape=jax.ShapeDtypeStruct((K * C, H, W), jnp.float32),
        scratch_shapes=[
            pltpu.VMEM((O, H, W), jnp.float32),
            pltpu.VMEM((H + 14, W), jnp.float32),
        ],
    )(xe2, ye2, x2, lt2)

    out = jnp.concatenate([x0, z], axis=0)[None]
    return out


def kernel(x, xe, ye, log_temp):
    return _run(x, xe, ye, log_temp)


# speedup vs baseline: 3.9210x; 1.0015x over previous
import functools
import jax
import jax.numpy as jnp
from jax import lax
from jax.experimental import pallas as pl
from jax.experimental.pallas import tpu as pltpu

K = 7
PS = 3
WS = 15
EPS = 1e-8
R = WS // 2
O = WS * WS
H = W = 128
CE = 8
C = 3
NEG_BIG = -1.0e30


def _rollw(a, s):
    if s % W == 0:
        return a
    return jnp.roll(a, s, axis=-1)


def _boxw(a):
    return a + _rollw(a, 1) + _rollw(a, -1)


def _nnn_kernel(xe2_ref, ye2_ref, x2_ref, lt_ref, out_ref, u_ref, bx_ref):
    f32 = jnp.float32

    lt = lt_ref[0]
    blt = _boxw(lt[0:H] + lt[1:H + 1] + lt[2:H + 2])
    invt = -1.0 / (jnp.exp(blt * (1.0 / (PS * PS))) + EPS)

    ny = jnp.zeros((H + 2, W), f32)
    for c in range(CE):
        yc = ye2_ref[c]
        ny = ny + yc * yc
    by = _boxw(ny[0:H] + ny[1:H + 1] + ny[2:H + 2])

    nx = jnp.zeros((H + 16, W), f32)
    for c in range(CE):
        xc = xe2_ref[c]
        nx = nx + xc * xc
    bx_ref[...] = _boxw(nx[0:H + 14] + nx[1:H + 15] + nx[2:H + 16])

    def dy_body(i, m):
        xh = [xe2_ref[c, pl.ds(i, H + 2), :] for c in range(CE)]
        yv = [ye2_ref[c] for c in range(CE)]
        bxs_rows = bx_ref[pl.ds(i, H), :]
        for dxk in range(WS):
            dx = dxk - R
            cc = jnp.zeros((H + 2, W), f32)
            for c in range(CE):
                cc = cc + yv[c] * _rollw(xh[c], -dx)
            bc = _boxw(cc[0:H] + cc[1:H + 1] + cc[2:H + 2])
            d = by + _rollw(bxs_rows, -dx) - 2.0 * bc
            lg = d * invt
            if dxk == R:
                lg = jnp.where(i == R, NEG_BIG, lg)
            u_ref[pl.ds(i * WS + dxk, 1)] = lg[None]
            m = jnp.maximum(m, lg)
        return m

    m = lax.fori_loop(0, WS, dy_body, jnp.full((H, W), -3.0e38, f32))

    def exp_body(o, _):
        u_ref[pl.ds(o, 1)] = jnp.exp(u_ref[pl.ds(o, 1)] - m[None])
        return 0

    lax.fori_loop(0, O, exp_body, 0)

    def sum_body(o, s):
        return s + u_ref[pl.ds(o, 1)][0]

    for j in range(K):
        s = lax.fori_loop(0, O, sum_body, jnp.zeros((H, W), f32))

        z0 = jnp.zeros((H, W), f32)
        zacc = [z0, z0, z0]
        for dxk in range(WS):
            dx = dxk - R

            def agg_body(i, accs, dxk=dxk, dx=dx):
                o = i * WS + dxk
                u = u_ref[pl.ds(o, 1)][0]
                w = u / s
                u_ref[pl.ds(o, 1)] = (u * (1.0 + (EPS - w)))[None]
                ws = _rollw(w, dx)
                a0, a1, a2 = accs
                a0 = a0 + ws * x2_ref[0, pl.ds(i, H), :]
                a1 = a1 + ws * x2_ref[1, pl.ds(i, H), :]
                a2 = a2 + ws * x2_ref[2, pl.ds(i, H), :]
                return (a0, a1, a2)

            acc = lax.fori_loop(0, WS, agg_body, (z0, z0, z0))
            for c in range(C):
                zacc[c] = zacc[c] + _rollw(acc[c], -dx)
        for c in range(C):
            out_ref[j * C + c] = zacc[c] - x2_ref[c, pl.ds(R, H), :]


@jax.jit
def _run(x, xe, ye, log_temp):
    x0 = x[0]
    xe0 = xe[0]
    ye0 = ye[0]
    lt0 = log_temp[0]

    xe2 = jnp.concatenate([xe0[:, -8:, :], xe0, xe0[:, :8, :]], axis=1)
    ye2 = jnp.concatenate([ye0[:, -1:, :], ye0, ye0[:, :1, :]], axis=1)
    x2 = jnp.concatenate([x0[:, -R:, :], x0, x0[:, :R, :]], axis=1)
    lt2 = jnp.concatenate([lt0[:, -1:, :], lt0, lt0[:, :1, :]], axis=1)

    z = pl.pallas_call(
        _nnn_kernel,
        out_shape=jax.ShapeDtypeStruct((K * C, H, W), jnp.float32),
        scratch_shapes=[
            pltpu.VMEM((O, H, W), jnp.float32),
            pltpu.VMEM((H + 14, W), jnp.float32),
        ],
    )(xe2, ye2, x2, lt2)

    out = jnp.concatenate([x0, z], axis=0)[None]
    return out


def kernel(x, xe, ye, log_temp):
    return _run(x, xe, ye, log_temp)


# speedup vs baseline: 5.7723x; 1.4721x over previous
import functools
import jax
import jax.numpy as jnp
from jax import lax
from jax.experimental import pallas as pl
from jax.experimental.pallas import tpu as pltpu

K = 7
PS = 3
WS = 15
EPS = 1e-8
R = WS // 2
O = WS * WS
H = W = 128
CE = 8
C = 3
NEG_BIG = -1.0e30


def _rollw(a, s):
    if s % W == 0:
        return a
    return jnp.roll(a, s, axis=-1)


def _boxw(a):
    return a + _rollw(a, 1) + _rollw(a, -1)


def _nnn_kernel(xe2_ref, ye2_ref, x2_ref, lt_ref, out_ref, u_ref, bx_ref):
    f32 = jnp.float32

    lt = lt_ref[0]
    blt = _boxw(lt[0:H] + lt[1:H + 1] + lt[2:H + 2])
    invt = -1.0 / (jnp.exp(blt * (1.0 / (PS * PS))) + EPS)

    ny = jnp.zeros((H + 2, W), f32)
    for c in range(CE):
        yc = ye2_ref[c]
        ny = ny + yc * yc
    by = _boxw(ny[0:H] + ny[1:H + 1] + ny[2:H + 2])

    nx = jnp.zeros((H + 16, W), f32)
    for c in range(CE):
        xc = xe2_ref[c]
        nx = nx + xc * xc
    bx_ref[...] = _boxw(nx[0:H + 14] + nx[1:H + 15] + nx[2:H + 16])

    def dy_body(i, m):
        xh = [xe2_ref[c, pl.ds(i, H + 2), :] for c in range(CE)]
        yv = [ye2_ref[c] for c in range(CE)]
        bxs_rows = bx_ref[pl.ds(i, H), :]
        for dxk in range(WS):
            dx = dxk - R
            cc = jnp.zeros((H + 2, W), f32)
            for c in range(CE):
                cc = cc + yv[c] * _rollw(xh[c], -dx)
            bc = _boxw(cc[0:H] + cc[1:H + 1] + cc[2:H + 2])
            d = by + _rollw(bxs_rows, -dx) - 2.0 * bc
            lg = d * invt
            if dxk == R:
                lg = jnp.where(i == R, NEG_BIG, lg)
            u_ref[pl.ds(i * WS + dxk, 1)] = lg[None]
            m = jnp.maximum(m, lg)
        return m

    m = lax.fori_loop(0, WS, dy_body, jnp.full((H, W), -3.0e38, f32))

    def exp_body(o, s):
        u = jnp.exp(u_ref[pl.ds(o, 1)] - m[None])
        u_ref[pl.ds(o, 1)] = u
        return s + u[0]

    s = lax.fori_loop(0, O, exp_body, jnp.zeros((H, W), f32))

    for j in range(K):
        last = j == K - 1

        def agg_body(i, accs, last=last):
            xh = [x2_ref[c, pl.ds(i, H), :] for c in range(C)]
            a0, a1, a2, sn = accs
            for dxk in range(WS):
                dx = dxk - R
                o = i * WS + dxk
                u = u_ref[pl.ds(o, 1)][0]
                w = u / s
                a0 = a0 + w * _rollw(xh[0], -dx)
                a1 = a1 + w * _rollw(xh[1], -dx)
                a2 = a2 + w * _rollw(xh[2], -dx)
                if not last:
                    un = u * (1.0 + (EPS - w))
                    u_ref[pl.ds(o, 1)] = un[None]
                    sn = sn + un
            return (a0, a1, a2, sn)

        z = jnp.zeros((H, W), f32)
        acc = lax.fori_loop(0, WS, agg_body, (z, z, z, z))
        for c in range(C):
            out_ref[j * C + c] = acc[c] - x2_ref[c, pl.ds(R, H), :]
        s = acc[3]


@jax.jit
def _run(x, xe, ye, log_temp):
    x0 = x[0]
    xe0 = xe[0]
    ye0 = ye[0]
    lt0 = log_temp[0]

    xe2 = jnp.concatenate([xe0[:, -8:, :], xe0, xe0[:, :8, :]], axis=1)
    ye2 = jnp.concatenate([ye0[:, -1:, :], ye0, ye0[:, :1, :]], axis=1)
    x2 = jnp.concatenate([x0[:, -R:, :], x0, x0[:, :R, :]], axis=1)
    lt2 = jnp.concatenate([lt0[:, -1:, :], lt0, lt0[:, :1, :]], axis=1)

    z = pl.pallas_call(
        _nnn_kernel,
        out_shape=jax.ShapeDtypeStruct((K * C, H, W), jnp.float32),
        scratch_shapes=[
            pltpu.VMEM((O, H, W), jnp.float32),
            pltpu.VMEM((H + 14, W), jnp.float32),
        ],
    )(xe2, ye2, x2, lt2)

    out = jnp.concatenate([x0, z], axis=0)[None]
    return out


def kernel(x, xe, ye, log_temp):
    return _run(x, xe, ye, log_temp)
